# Initial kernel scaffold; baseline (speedup 1.0000x reference)
import jax
import jax.numpy as jnp
from jax.experimental import pallas as pl


def kernel(x, edge_index, edge_attr, W1, b1, Wl, bl):
    raise NotImplementedError("write your pallas kernel here")


# speedup vs baseline: 14.5243x; 14.5243x over previous
import functools

import jax
import jax.numpy as jnp
from jax import lax
from jax.experimental import pallas as pl
from jax.experimental.pallas import tpu as pltpu
from jax.experimental.pallas import tpu_sc as plsc

N_NODES = 10000
N_PAD = 10240
E = 320000
D = 128
L = 16
NC = 2
NS = 16
CH = 128
NCH = E // CH
ROWS_PER_TILE = N_PAD // NS


def _zero_vec16():
    return jnp.zeros((L,), jnp.float32)


def _sc_body(row_hbm, col_hbm, ea_hbm, y_hbm,
             agg_hbm, dis2_hbm,
             sh_agg, sh_deg, sh_dis,
             gbuf, dis_full, deg_loc, dis_loc, dis2_loc, z640,
             ridx, cidx, eab, coeffb, onesb, sem):
    c = lax.axis_index("c")
    s = lax.axis_index("s")
    wid = c * NS + s

    def _zero_gbuf(i, _):
        for j in range(D // L):
            gbuf[i, pl.ds(j * L, L)] = _zero_vec16()
        return 0
    lax.fori_loop(0, CH, _zero_gbuf, 0)

    def _zero_z640(i, _):
        z640[pl.ds(i * L, L)] = _zero_vec16()
        return 0
    lax.fori_loop(0, ROWS_PER_TILE // L, _zero_z640, 0)

    def _ones(i, _):
        onesb[pl.ds(i * L, L)] = jnp.full((L,), 1.0, jnp.float32)
        return 0
    lax.fori_loop(0, CH // L, _ones, 0)

    base_n = s * ROWS_PER_TILE
    for t in range(ROWS_PER_TILE // CH):
        pltpu.sync_copy(gbuf, sh_agg.at[pl.ds(base_n + t * CH, CH)])
    pltpu.sync_copy(z640, sh_deg.at[pl.ds(base_n, ROWS_PER_TILE)])
    plsc.subcore_barrier()

    def _deg_chunk(i, _):
        k = s + NS * i

        @pl.when(k < NCH)
        def _():
            pltpu.sync_copy(row_hbm.at[pl.ds(k * CH, CH)], ridx)
            pltpu.sync_copy(onesb, sh_deg.at[ridx], add=True)
        return 0
    lax.fori_loop(0, (NCH + NS - 1) // NS, _deg_chunk, 0)
    plsc.subcore_barrier()

    pltpu.sync_copy(sh_deg.at[pl.ds(base_n, ROWS_PER_TILE)], deg_loc)

    def _rsqrt(i, _):
        sl = pl.ds(i * L, L)
        v = deg_loc[sl] + 1.0
        bits = plsc.bitcast(v, jnp.int32)
        g = jnp.int32(0x5F3759DF) - lax.shift_right_arithmetic(bits, 1)
        yv = plsc.bitcast(g, jnp.float32)
        for _ in range(3):
            yv = yv * (1.5 - 0.5 * v * yv * yv)
        dis_loc[sl] = yv
        dis2_loc[sl] = yv * yv
        return 0
    lax.fori_loop(0, ROWS_PER_TILE // L, _rsqrt, 0)

    pltpu.sync_copy(dis_loc, sh_dis.at[pl.ds(base_n, ROWS_PER_TILE)])

    @pl.when(c == 0)
    def _():
        pltpu.sync_copy(dis2_loc, dis2_hbm.at[pl.ds(base_n, ROWS_PER_TILE)])
    plsc.subcore_barrier()

    pltpu.sync_copy(sh_dis, dis_full)

    def _edge_chunk(i, _):
        k = wid + NC * NS * i

        @pl.when(k < NCH)
        def _():
            base = k * CH
            pltpu.sync_copy(row_hbm.at[pl.ds(base, CH)], ridx)
            pltpu.sync_copy(col_hbm.at[pl.ds(base, CH)], cidx)
            pltpu.sync_copy(ea_hbm.at[pl.ds(base, CH)], eab)
            gather = pltpu.async_copy(y_hbm.at[ridx], gbuf, sem)

            def _coeff(j, _c):
                sl = pl.ds(j * L, L)
                dr = plsc.load_gather(dis_full, [ridx[sl]])
                dc = plsc.load_gather(dis_full, [cidx[sl]])
                coeffb[sl] = eab[sl] * dr * dc
                return 0
            lax.fori_loop(0, CH // L, _coeff, 0)
            gather.wait()

            def _scale(e, _c):
                sp = plsc.load_gather(coeffb, [jnp.full((L,), e, jnp.int32)])
                for j in range(D // L):
                    sl = pl.ds(j * L, L)
                    gbuf[e, sl] = gbuf[e, sl] * sp
                return 0
            lax.fori_loop(0, CH, _scale, 0)

            pltpu.sync_copy(gbuf, sh_agg.at[cidx], add=True)
        return 0
    lax.fori_loop(0, (NCH + NC * NS - 1) // (NC * NS), _edge_chunk, 0)
    plsc.subcore_barrier()

    pltpu.sync_copy(sh_agg.at[pl.ds(base_n, ROWS_PER_TILE)],
                    agg_hbm.at[c, pl.ds(base_n, ROWS_PER_TILE)])


def _sc_aggregate(row, col, ea, y):
    mesh = plsc.VectorSubcoreMesh(core_axis_name="c", subcore_axis_name="s",
                                  num_cores=NC, num_subcores=NS)
    f = pl.kernel(
        _sc_body,
        out_type=(jax.ShapeDtypeStruct((NC, N_PAD, D), jnp.float32),
                  jax.ShapeDtypeStruct((N_PAD,), jnp.float32)),
        mesh=mesh,
        scratch_types=[
            pltpu.VMEM_SHARED((N_PAD, D), jnp.float32),
            pltpu.VMEM_SHARED((N_PAD,), jnp.float32),
            pltpu.VMEM_SHARED((N_PAD,), jnp.float32),
            pltpu.VMEM((CH, D), jnp.float32),
            pltpu.VMEM((N_PAD,), jnp.float32),
            pltpu.VMEM((ROWS_PER_TILE,), jnp.float32),
            pltpu.VMEM((ROWS_PER_TILE,), jnp.float32),
            pltpu.VMEM((ROWS_PER_TILE,), jnp.float32),
            pltpu.VMEM((ROWS_PER_TILE,), jnp.float32),
            pltpu.VMEM((CH,), jnp.int32),
            pltpu.VMEM((CH,), jnp.int32),
            pltpu.VMEM((CH,), jnp.float32),
            pltpu.VMEM((CH,), jnp.float32),
            pltpu.VMEM((CH,), jnp.float32),
            pltpu.SemaphoreType.DMA,
        ],
        compiler_params=pltpu.CompilerParams(needs_layout_passes=False),
    )
    return f(row, col, ea, y)


_BM = 1000


def _matmul_body(x_ref, w_ref, o_ref):
    o_ref[...] = jnp.dot(x_ref[...], w_ref[...],
                         preferred_element_type=jnp.float32)


def _tc_matmul(x, W1):
    return pl.pallas_call(
        _matmul_body,
        grid=(N_NODES // _BM,),
        in_specs=[pl.BlockSpec((_BM, D), lambda i: (i, 0)),
                  pl.BlockSpec((D, D), lambda i: (0, 0))],
        out_specs=pl.BlockSpec((_BM, D), lambda i: (i, 0)),
        out_shape=jax.ShapeDtypeStruct((N_NODES, D), jnp.float32),
    )(x, W1)


def _epilogue_body(a0_ref, a1_ref, y_ref, d2_ref, b1_ref, wl_ref, bl_ref, o_ref):
    a = (a0_ref[...] + a1_ref[...]
         + y_ref[...] * d2_ref[...]
         + b1_ref[...])
    h = jnp.tanh(a)
    o_ref[...] = jax.nn.sigmoid(
        jnp.dot(h, wl_ref[...], preferred_element_type=jnp.float32)
        + bl_ref[0, 0])


def _tc_epilogue(agg, y, dis2, b1, Wl, bl):
    a0 = agg[0]
    a1 = agg[1]
    d2 = dis2[:, None]
    b1r = b1[None, :]
    blr = bl[None, :]
    return pl.pallas_call(
        _epilogue_body,
        grid=(N_NODES // _BM,),
        in_specs=[pl.BlockSpec((_BM, D), lambda i: (i, 0)),
                  pl.BlockSpec((_BM, D), lambda i: (i, 0)),
                  pl.BlockSpec((_BM, D), lambda i: (i, 0)),
                  pl.BlockSpec((_BM, 1), lambda i: (i, 0)),
                  pl.BlockSpec((1, D), lambda i: (0, 0)),
                  pl.BlockSpec((D, 1), lambda i: (0, 0)),
                  pl.BlockSpec((1, 1), lambda i: (0, 0))],
        out_specs=pl.BlockSpec((_BM, 1), lambda i: (i, 0)),
        out_shape=jax.ShapeDtypeStruct((N_NODES, 1), jnp.float32),
    )(a0, a1, y, d2, b1r, Wl, blr)


def kernel(x, edge_index, edge_attr, W1, b1, Wl, bl):
    x = x.astype(jnp.float32)
    ei = edge_index.astype(jnp.int32)
    row = ei[0]
    col = ei[1]
    ea = edge_attr.astype(jnp.float32)

    y = _tc_matmul(x, W1.astype(jnp.float32))
    agg, dis2 = _sc_aggregate(row, col, ea, y)
    out = _tc_epilogue(agg, y, dis2[:N_NODES], b1.astype(jnp.float32),
                       Wl.astype(jnp.float32), bl.astype(jnp.float32))
    return out


# speedup vs baseline: 27.3733x; 1.8847x over previous
import jax
import jax.numpy as jnp
from jax import lax
from jax.experimental import pallas as pl
from jax.experimental.pallas import tpu as pltpu
from jax.experimental.pallas import tpu_sc as plsc

N_NODES = 10000
N_PAD = 10240
E = 320000
D = 128
L = 16
NC = 2
NS = 16
NW = NC * NS
CH = 64
NCH = 5120
CPW = NCH // NW
NBUF = 4
ROWS_PER_TILE = N_PAD // NS


def _zero_vec16():
    return jnp.zeros((L,), jnp.float32)


def _deg_body(rowd_hbm, deg_hbm, sh_deg, stage, onesb, z640, sem1):
    c = lax.axis_index("c")
    s = lax.axis_index("s")
    wid = c * NS + s

    def _z(i, _):
        z640[pl.ds(i * L, L)] = _zero_vec16()
        return 0
    lax.fori_loop(0, ROWS_PER_TILE // L, _z, 0)

    def _o(i, _):
        onesb[pl.ds(i * L, L)] = jnp.full((L,), 1.0, jnp.float32)
        return 0
    lax.fori_loop(0, CH // L, _o, 0)

    base_n = s * ROWS_PER_TILE
    pltpu.sync_copy(z640, sh_deg.at[pl.ds(base_n, ROWS_PER_TILE)])
    plsc.subcore_barrier()

    pltpu.sync_copy(rowd_hbm.at[pl.ds(wid * CPW, CPW)], stage)

    def _grp(g, _):
        for u in range(8):
            pltpu.async_copy(onesb, sh_deg.at[stage.at[g * 8 + u]],
                             sem1, add=True)
        for u in range(8):
            pltpu.make_async_copy(onesb, sh_deg.at[stage.at[g * 8 + u]],
                                  sem1).wait()
        return 0
    lax.fori_loop(0, CPW // 8, _grp, 0)
    plsc.subcore_barrier()

    pltpu.sync_copy(sh_deg.at[pl.ds(base_n, ROWS_PER_TILE)],
                    deg_hbm.at[c, pl.ds(base_n, ROWS_PER_TILE)])


def _sc_degree(rowd):
    mesh = plsc.VectorSubcoreMesh(core_axis_name="c", subcore_axis_name="s",
                                  num_cores=NC, num_subcores=NS)
    f = pl.kernel(
        _deg_body,
        out_type=jax.ShapeDtypeStruct((NC, N_PAD), jnp.float32),
        mesh=mesh,
        scratch_types=[
            pltpu.VMEM_SHARED((N_PAD,), jnp.float32),
            pltpu.VMEM((CPW, CH), jnp.int32),
            pltpu.VMEM((CH,), jnp.float32),
            pltpu.VMEM((ROWS_PER_TILE,), jnp.float32),
            pltpu.SemaphoreType.DMA,
        ],
        compiler_params=pltpu.CompilerParams(needs_layout_passes=False),
    )
    return f(rowd)


def _edge_body(rowg_hbm, col_hbm, ea_hbm, y_hbm, dis_hbm,
               agg_hbm,
               sh_agg, gb0, gb1, gb2, gb3,
               dis_full, ridx4, cidx4, ea4, co4,
               semi, semg, semsc):
    c = lax.axis_index("c")
    s = lax.axis_index("s")
    wid = c * NS + s
    gbufs = (gb0, gb1, gb2, gb3)
    cbase = wid * CPW

    def _zero_gb0(i, _):
        for j in range(D // L):
            gb0[i, pl.ds(j * L, L)] = _zero_vec16()
        return 0
    lax.fori_loop(0, CH, _zero_gb0, 0)

    base_n = s * ROWS_PER_TILE
    for t in range(ROWS_PER_TILE // CH):
        pltpu.sync_copy(gb0, sh_agg.at[pl.ds(base_n + t * CH, CH)])
    pltpu.sync_copy(dis_hbm, dis_full)
    plsc.subcore_barrier()

    def _start_idx(slot, k):
        pltpu.async_copy(rowg_hbm.at[cbase + k], ridx4.at[slot], semi.at[slot])
        pltpu.async_copy(col_hbm.at[cbase + k], cidx4.at[slot], semi.at[slot])
        pltpu.async_copy(ea_hbm.at[cbase + k], ea4.at[slot], semi.at[slot])

    def _wait_idx(slot, k):
        pltpu.make_async_copy(rowg_hbm.at[cbase + k], ridx4.at[slot],
                              semi.at[slot]).wait()
        pltpu.make_async_copy(col_hbm.at[cbase + k], cidx4.at[slot],
                              semi.at[slot]).wait()
        pltpu.make_async_copy(ea_hbm.at[cbase + k], ea4.at[slot],
                              semi.at[slot]).wait()

    def _start_gather(slot, k):
        pltpu.async_copy(y_hbm.at[ridx4.at[slot]], gbufs[slot], semg.at[slot])

    def _wait_gather(slot):
        pltpu.make_async_copy(y_hbm.at[ridx4.at[slot]], gbufs[slot],
                              semg.at[slot]).wait()

    def _start_scatter(slot):
        pltpu.async_copy(gbufs[slot], sh_agg.at[cidx4.at[slot]],
                         semsc.at[slot], add=True)

    def _wait_scatter(slot):
        pltpu.make_async_copy(gbufs[slot], sh_agg.at[cidx4.at[slot]],
                              semsc.at[slot]).wait()

    _start_idx(0, 0)
    _start_idx(1, 1)
    _wait_idx(0, 0)
    _start_gather(0, 0)

    def _outer(o, _):
        for u in range(NBUF):
            i = o * NBUF + u
            nslot = (u + 1) % NBUF
            fslot = (u + 2) % NBUF

            @pl.when(i + 1 < CPW)
            def _():
                _wait_idx(nslot, i + 1)
                _start_gather(nslot, i + 1)

            _wait_gather(u)

            def _coeff(j, _c):
                sl = pl.ds(j * L, L)
                dr = plsc.load_gather(dis_full, [ridx4[u, sl]])
                dc = plsc.load_gather(dis_full, [cidx4[u, sl]])
                co4[u, sl] = ea4[u, sl] * dr * dc
                return 0
            lax.fori_loop(0, CH // L, _coeff, 0)

            gb = gbufs[u]

            def _scale(e, _c):
                sp = plsc.load_gather(
                    co4, [jnp.full((L,), u, jnp.int32),
                          jnp.full((L,), e, jnp.int32)])
                for j in range(D // L):
                    sl = pl.ds(j * L, L)
                    gb[e, sl] = gb[e, sl] * sp
                return 0
            lax.fori_loop(0, CH, _scale, 0)

            _start_scatter(u)

            @pl.when(i >= 2)
            def _():
                _wait_scatter(fslot)

            @pl.when(i + 2 < CPW)
            def _():
                _start_idx(fslot, i + 2)
        return 0
    lax.fori_loop(0, CPW // NBUF, _outer, 0)

    _wait_scatter((CPW - 2) % NBUF)
    _wait_scatter((CPW - 1) % NBUF)
    plsc.subcore_barrier()

    pltpu.sync_copy(sh_agg.at[pl.ds(base_n, ROWS_PER_TILE)],
                    agg_hbm.at[c, pl.ds(base_n, ROWS_PER_TILE)])


def _sc_edges(rowg, col, ea, y, dis):
    mesh = plsc.VectorSubcoreMesh(core_axis_name="c", subcore_axis_name="s",
                                  num_cores=NC, num_subcores=NS)
    f = pl.kernel(
        _edge_body,
        out_type=jax.ShapeDtypeStruct((NC, N_PAD, D), jnp.float32),
        mesh=mesh,
        scratch_types=[
            pltpu.VMEM_SHARED((N_PAD, D), jnp.float32),
            pltpu.VMEM((CH, D), jnp.float32),
            pltpu.VMEM((CH, D), jnp.float32),
            pltpu.VMEM((CH, D), jnp.float32),
            pltpu.VMEM((CH, D), jnp.float32),
            pltpu.VMEM((N_PAD,), jnp.float32),
            pltpu.VMEM((NBUF, CH), jnp.int32),
            pltpu.VMEM((NBUF, CH), jnp.int32),
            pltpu.VMEM((NBUF, CH), jnp.float32),
            pltpu.VMEM((NBUF, CH), jnp.float32),
            pltpu.SemaphoreType.DMA((NBUF,)),
            pltpu.SemaphoreType.DMA((NBUF,)),
            pltpu.SemaphoreType.DMA((NBUF,)),
        ],
        compiler_params=pltpu.CompilerParams(needs_layout_passes=False),
    )
    return f(rowg, col, ea, y, dis)


_BM = 1000


def _matmul_body(x_ref, w_ref, o_ref):
    o_ref[...] = jnp.dot(x_ref[...], w_ref[...],
                         preferred_element_type=jnp.float32)


def _tc_matmul(x, W1):
    return pl.pallas_call(
        _matmul_body,
        grid=(N_NODES // _BM,),
        in_specs=[pl.BlockSpec((_BM, D), lambda i: (i, 0)),
                  pl.BlockSpec((D, D), lambda i: (0, 0))],
        out_specs=pl.BlockSpec((_BM, D), lambda i: (i, 0)),
        out_shape=jax.ShapeDtypeStruct((N_NODES, D), jnp.float32),
    )(x, W1)


def _dis_body(deg_ref, dis_ref, dis2_ref):
    d = deg_ref[0] + deg_ref[1] + 1.0
    dis_ref[...] = lax.rsqrt(d)
    dis2_ref[...] = 1.0 / d


def _tc_dis(deg2):
    return pl.pallas_call(
        _dis_body,
        out_shape=(jax.ShapeDtypeStruct((N_PAD // D, D), jnp.float32),
                   jax.ShapeDtypeStruct((N_PAD // D, D), jnp.float32)),
    )(deg2)


def _epilogue_body(a0_ref, a1_ref, y_ref, d2_ref, b1_ref, wl_ref, bl_ref, o_ref):
    a = (a0_ref[...] + a1_ref[...]
         + y_ref[...] * d2_ref[...]
         + b1_ref[...])
    h = jnp.tanh(a)
    o_ref[...] = jax.nn.sigmoid(
        jnp.dot(h, wl_ref[...], preferred_element_type=jnp.float32)
        + bl_ref[0, 0])


def _tc_epilogue(agg, y, dis2, b1, Wl, bl):
    a0 = agg[0]
    a1 = agg[1]
    d2 = dis2[:, None]
    b1r = b1[None, :]
    blr = bl[None, :]
    return pl.pallas_call(
        _epilogue_body,
        grid=(N_NODES // _BM,),
        in_specs=[pl.BlockSpec((_BM, D), lambda i: (i, 0)),
                  pl.BlockSpec((_BM, D), lambda i: (i, 0)),
                  pl.BlockSpec((_BM, D), lambda i: (i, 0)),
                  pl.BlockSpec((_BM, 1), lambda i: (i, 0)),
                  pl.BlockSpec((1, D), lambda i: (0, 0)),
                  pl.BlockSpec((D, 1), lambda i: (0, 0)),
                  pl.BlockSpec((1, 1), lambda i: (0, 0))],
        out_specs=pl.BlockSpec((_BM, 1), lambda i: (i, 0)),
        out_shape=jax.ShapeDtypeStruct((N_NODES, 1), jnp.float32),
    )(a0, a1, y, d2, b1r, Wl, blr)


def kernel(x, edge_index, edge_attr, W1, b1, Wl, bl):
    x = x.astype(jnp.float32)
    ei = edge_index.astype(jnp.int32)
    row = ei[0]
    col = ei[1]
    ea = edge_attr.astype(jnp.float32)

    npad = NCH * CH - E
    arp = jnp.arange(npad, dtype=jnp.int32)
    pad_hi = N_NODES + (arp % (N_PAD - N_NODES))
    rowd = jnp.concatenate([row, pad_hi]).reshape(NCH, CH)
    rowg = jnp.concatenate([row, arp % N_NODES]).reshape(NCH, CH)
    colp = jnp.concatenate([col, pad_hi]).reshape(NCH, CH)
    eap = jnp.concatenate([ea, jnp.zeros((npad,), jnp.float32)]).reshape(NCH, CH)

    deg2 = _sc_degree(rowd)
    y = _tc_matmul(x, W1.astype(jnp.float32))
    dis_t, dis2_t = _tc_dis(deg2.reshape(NC, N_PAD // D, D))
    dis = dis_t.reshape(N_PAD)
    agg = _sc_edges(rowg, colp, eap, y, dis)
    out = _tc_epilogue(agg, y, dis2_t.reshape(N_PAD)[:N_NODES],
                       b1.astype(jnp.float32), Wl.astype(jnp.float32),
                       bl.astype(jnp.float32))
    return out


# speedup vs baseline: 31.4557x; 1.1491x over previous
import jax
import jax.numpy as jnp
from jax import lax
from jax.experimental import pallas as pl
from jax.experimental.pallas import tpu as pltpu
from jax.experimental.pallas import tpu_sc as plsc

N_NODES = 10000
N_PAD = 10240
E = 320000
D = 128
L = 16
NC = 2
NS = 16
NW = NC * NS
CH = 64
NCH = 5120
CPW = NCH // NW
NBUF = 4
ROWS_PER_TILE = N_PAD // NS


def _zero_vec16():
    return jnp.zeros((L,), jnp.float32)


def _deg_body(rowd_hbm, deg_hbm, sh_deg, stage, onesb, z640, sem1):
    c = lax.axis_index("c")
    s = lax.axis_index("s")
    wid = c * NS + s

    def _z(i, _):
        z640[pl.ds(i * L, L)] = _zero_vec16()
        return 0
    lax.fori_loop(0, ROWS_PER_TILE // L, _z, 0)

    def _o(i, _):
        onesb[pl.ds(i * L, L)] = jnp.full((L,), 1.0, jnp.float32)
        return 0
    lax.fori_loop(0, CH // L, _o, 0)

    base_n = s * ROWS_PER_TILE
    pltpu.sync_copy(z640, sh_deg.at[pl.ds(base_n, ROWS_PER_TILE)])
    plsc.subcore_barrier()

    pltpu.sync_copy(rowd_hbm.at[pl.ds(wid * CPW, CPW)], stage)

    def _grp(g, _):
        for u in range(8):
            pltpu.async_copy(onesb, sh_deg.at[stage.at[g * 8 + u]],
                             sem1, add=True)
        for u in range(8):
            pltpu.make_async_copy(onesb, sh_deg.at[stage.at[g * 8 + u]],
                                  sem1).wait()
        return 0
    lax.fori_loop(0, CPW // 8, _grp, 0)
    plsc.subcore_barrier()

    pltpu.sync_copy(sh_deg.at[pl.ds(base_n, ROWS_PER_TILE)],
                    deg_hbm.at[c, pl.ds(base_n, ROWS_PER_TILE)])


def _sc_degree(rowd):
    mesh = plsc.VectorSubcoreMesh(core_axis_name="c", subcore_axis_name="s",
                                  num_cores=NC, num_subcores=NS)
    f = pl.kernel(
        _deg_body,
        out_type=jax.ShapeDtypeStruct((NC, N_PAD), jnp.float32),
        mesh=mesh,
        scratch_types=[
            pltpu.VMEM_SHARED((N_PAD,), jnp.float32),
            pltpu.VMEM((CPW, CH), jnp.int32),
            pltpu.VMEM((CH,), jnp.float32),
            pltpu.VMEM((ROWS_PER_TILE,), jnp.float32),
            pltpu.SemaphoreType.DMA,
        ],
        compiler_params=pltpu.CompilerParams(needs_layout_passes=False),
    )
    return f(rowd)


def _edge_body(rowg_hbm, col_hbm, ea_hbm, y_hbm, dis_hbm,
               agg_hbm,
               sh_agg, gb0, gb1, gb2, gb3,
               dis_full, ridx4, cidx4, ea4, co4,
               semi, semg, semsc):
    c = lax.axis_index("c")
    s = lax.axis_index("s")
    wid = c * NS + s
    gbufs = (gb0, gb1, gb2, gb3)
    cbase = wid * CPW

    def _zero_gb0(i, _):
        for j in range(D // L):
            gb0[i, pl.ds(j * L, L)] = _zero_vec16()
        return 0
    lax.fori_loop(0, CH, _zero_gb0, 0)

    base_n = s * ROWS_PER_TILE
    for t in range(ROWS_PER_TILE // CH):
        pltpu.sync_copy(gb0, sh_agg.at[pl.ds(base_n + t * CH, CH)])
    pltpu.sync_copy(dis_hbm, dis_full)
    plsc.subcore_barrier()

    def _start_idx(slot, k):
        pltpu.async_copy(rowg_hbm.at[cbase + k], ridx4.at[slot], semi.at[slot])
        pltpu.async_copy(col_hbm.at[cbase + k], cidx4.at[slot], semi.at[slot])
        pltpu.async_copy(ea_hbm.at[cbase + k], ea4.at[slot], semi.at[slot])

    def _wait_idx(slot, k):
        pltpu.make_async_copy(rowg_hbm.at[cbase + k], ridx4.at[slot],
                              semi.at[slot]).wait()
        pltpu.make_async_copy(col_hbm.at[cbase + k], cidx4.at[slot],
                              semi.at[slot]).wait()
        pltpu.make_async_copy(ea_hbm.at[cbase + k], ea4.at[slot],
                              semi.at[slot]).wait()

    def _start_gather(slot, k):
        pltpu.async_copy(y_hbm.at[ridx4.at[slot]], gbufs[slot], semg.at[slot])

    def _wait_gather(slot):
        pltpu.make_async_copy(y_hbm.at[ridx4.at[slot]], gbufs[slot],
                              semg.at[slot]).wait()

    def _start_scatter(slot):
        pltpu.async_copy(gbufs[slot], sh_agg.at[cidx4.at[slot]],
                         semsc.at[slot], add=True)

    def _wait_scatter(slot):
        pltpu.make_async_copy(gbufs[slot], sh_agg.at[cidx4.at[slot]],
                              semsc.at[slot]).wait()

    _start_idx(0, 0)
    _start_idx(1, 1)
    _wait_idx(0, 0)
    _start_gather(0, 0)

    def _outer(o, _):
        for u in range(NBUF):
            i = o * NBUF + u
            nslot = (u + 1) % NBUF
            fslot = (u + 2) % NBUF

            @pl.when(i + 1 < CPW)
            def _():
                _wait_idx(nslot, i + 1)
                _start_gather(nslot, i + 1)

            _wait_gather(u)

            def _coeff(j, _c):
                sl = pl.ds(j * L, L)
                dr = plsc.load_gather(dis_full, [ridx4[u, sl]])
                dc = plsc.load_gather(dis_full, [cidx4[u, sl]])
                co4[u, sl] = ea4[u, sl] * dr * dc
                return 0
            lax.fori_loop(0, CH // L, _coeff, 0)

            gb = gbufs[u]

            @plsc.parallel_loop(0, CH, 1, unroll=4)
            def _scale(e):
                sp = plsc.load_gather(
                    co4, [jnp.full((L,), u, jnp.int32),
                          jnp.full((L,), e, jnp.int32)])
                for j in range(D // L):
                    sl = pl.ds(j * L, L)
                    gb[e, sl] = gb[e, sl] * sp

            _start_scatter(u)

            @pl.when(i >= 2)
            def _():
                _wait_scatter(fslot)

            @pl.when(i + 2 < CPW)
            def _():
                _start_idx(fslot, i + 2)
        return 0
    lax.fori_loop(0, CPW // NBUF, _outer, 0)

    _wait_scatter((CPW - 2) % NBUF)
    _wait_scatter((CPW - 1) % NBUF)
    plsc.subcore_barrier()

    pltpu.sync_copy(sh_agg.at[pl.ds(base_n, ROWS_PER_TILE)],
                    agg_hbm.at[c, pl.ds(base_n, ROWS_PER_TILE)])


def _sc_edges(rowg, col, ea, y, dis):
    mesh = plsc.VectorSubcoreMesh(core_axis_name="c", subcore_axis_name="s",
                                  num_cores=NC, num_subcores=NS)
    f = pl.kernel(
        _edge_body,
        out_type=jax.ShapeDtypeStruct((NC, N_PAD, D), jnp.float32),
        mesh=mesh,
        scratch_types=[
            pltpu.VMEM_SHARED((N_PAD, D), jnp.float32),
            pltpu.VMEM((CH, D), jnp.float32),
            pltpu.VMEM((CH, D), jnp.float32),
            pltpu.VMEM((CH, D), jnp.float32),
            pltpu.VMEM((CH, D), jnp.float32),
            pltpu.VMEM((N_PAD,), jnp.float32),
            pltpu.VMEM((NBUF, CH), jnp.int32),
            pltpu.VMEM((NBUF, CH), jnp.int32),
            pltpu.VMEM((NBUF, CH), jnp.float32),
            pltpu.VMEM((NBUF, CH), jnp.float32),
            pltpu.SemaphoreType.DMA((NBUF,)),
            pltpu.SemaphoreType.DMA((NBUF,)),
            pltpu.SemaphoreType.DMA((NBUF,)),
        ],
        compiler_params=pltpu.CompilerParams(needs_layout_passes=False),
    )
    return f(rowg, col, ea, y, dis)


_BM = 1000


def _matmul_body(x_ref, w_ref, o_ref):
    o_ref[...] = jnp.dot(x_ref[...], w_ref[...],
                         preferred_element_type=jnp.float32)


def _tc_matmul(x, W1):
    return pl.pallas_call(
        _matmul_body,
        grid=(N_NODES // _BM,),
        in_specs=[pl.BlockSpec((_BM, D), lambda i: (i, 0)),
                  pl.BlockSpec((D, D), lambda i: (0, 0))],
        out_specs=pl.BlockSpec((_BM, D), lambda i: (i, 0)),
        out_shape=jax.ShapeDtypeStruct((N_NODES, D), jnp.float32),
    )(x, W1)


def _dis_body(deg_ref, dis_ref, dis2_ref):
    d = deg_ref[0] + deg_ref[1] + 1.0
    dis_ref[...] = lax.rsqrt(d)
    dis2_ref[...] = 1.0 / d


def _tc_dis(deg2):
    return pl.pallas_call(
        _dis_body,
        out_shape=(jax.ShapeDtypeStruct((N_PAD // D, D), jnp.float32),
                   jax.ShapeDtypeStruct((N_PAD // D, D), jnp.float32)),
    )(deg2)


def _epilogue_body(a0_ref, a1_ref, y_ref, d2_ref, b1_ref, wl_ref, bl_ref, o_ref):
    a = (a0_ref[...] + a1_ref[...]
         + y_ref[...] * d2_ref[...]
         + b1_ref[...])
    h = jnp.tanh(a)
    o_ref[...] = jax.nn.sigmoid(
        jnp.dot(h, wl_ref[...], preferred_element_type=jnp.float32)
        + bl_ref[0, 0])


def _tc_epilogue(agg, y, dis2, b1, Wl, bl):
    a0 = agg[0]
    a1 = agg[1]
    d2 = dis2[:, None]
    b1r = b1[None, :]
    blr = bl[None, :]
    return pl.pallas_call(
        _epilogue_body,
        grid=(N_NODES // _BM,),
        in_specs=[pl.BlockSpec((_BM, D), lambda i: (i, 0)),
                  pl.BlockSpec((_BM, D), lambda i: (i, 0)),
                  pl.BlockSpec((_BM, D), lambda i: (i, 0)),
                  pl.BlockSpec((_BM, 1), lambda i: (i, 0)),
                  pl.BlockSpec((1, D), lambda i: (0, 0)),
                  pl.BlockSpec((D, 1), lambda i: (0, 0)),
                  pl.BlockSpec((1, 1), lambda i: (0, 0))],
        out_specs=pl.BlockSpec((_BM, 1), lambda i: (i, 0)),
        out_shape=jax.ShapeDtypeStruct((N_NODES, 1), jnp.float32),
    )(a0, a1, y, d2, b1r, Wl, blr)


def kernel(x, edge_index, edge_attr, W1, b1, Wl, bl):
    x = x.astype(jnp.float32)
    ei = edge_index.astype(jnp.int32)
    row = ei[0]
    col = ei[1]
    ea = edge_attr.astype(jnp.float32)

    npad = NCH * CH - E
    arp = jnp.arange(npad, dtype=jnp.int32)
    pad_hi = N_NODES + (arp % (N_PAD - N_NODES))
    rowd = jnp.concatenate([row, pad_hi]).reshape(NCH, CH)
    rowg = jnp.concatenate([row, arp % N_NODES]).reshape(NCH, CH)
    colp = jnp.concatenate([col, pad_hi]).reshape(NCH, CH)
    eap = jnp.concatenate([ea, jnp.zeros((npad,), jnp.float32)]).reshape(NCH, CH)

    deg2 = _sc_degree(rowd)
    y = _tc_matmul(x, W1.astype(jnp.float32))
    dis_t, dis2_t = _tc_dis(deg2.reshape(NC, N_PAD // D, D))
    dis = dis_t.reshape(N_PAD)
    agg = _sc_edges(rowg, colp, eap, y, dis)
    out = _tc_epilogue(agg, y, dis2_t.reshape(N_PAD)[:N_NODES],
                       b1.astype(jnp.float32), Wl.astype(jnp.float32),
                       bl.astype(jnp.float32))
    return out


# speedup vs baseline: 33.6265x; 1.0690x over previous
import jax
import jax.numpy as jnp
from jax import lax
from jax.experimental import pallas as pl
from jax.experimental.pallas import tpu as pltpu
from jax.experimental.pallas import tpu_sc as plsc

N_NODES = 10000
N_PAD = 10240
E = 320000
D = 128
L = 16
NC = 2
NS = 16
NW = NC * NS
CHD = 64
NCHD = 5120
CPWD = NCHD // NW
CH = 80
NCH = 4096
CPW = NCH // NW
NBUF = 4
ROWS_PER_TILE = N_PAD // NS


def _zero_vec16():
    return jnp.zeros((L,), jnp.float32)


def _deg_body(rowd_hbm, deg_hbm, sh_deg, stage, onesb, z640, sem1):
    c = lax.axis_index("c")
    s = lax.axis_index("s")
    wid = c * NS + s

    def _z(i, _):
        z640[pl.ds(i * L, L)] = _zero_vec16()
        return 0
    lax.fori_loop(0, ROWS_PER_TILE // L, _z, 0)

    def _o(i, _):
        onesb[pl.ds(i * L, L)] = jnp.full((L,), 1.0, jnp.float32)
        return 0
    lax.fori_loop(0, CHD // L, _o, 0)

    base_n = s * ROWS_PER_TILE
    pltpu.sync_copy(z640, sh_deg.at[pl.ds(base_n, ROWS_PER_TILE)])
    plsc.subcore_barrier()

    pltpu.sync_copy(rowd_hbm.at[pl.ds(wid * CPWD, CPWD)], stage)

    def _grp(g, _):
        for u in range(8):
            pltpu.async_copy(onesb, sh_deg.at[stage.at[g * 8 + u]],
                             sem1, add=True)
        for u in range(8):
            pltpu.make_async_copy(onesb, sh_deg.at[stage.at[g * 8 + u]],
                                  sem1).wait()
        return 0
    lax.fori_loop(0, CPWD // 8, _grp, 0)
    plsc.subcore_barrier()

    pltpu.sync_copy(sh_deg.at[pl.ds(base_n, ROWS_PER_TILE)],
                    deg_hbm.at[c, pl.ds(base_n, ROWS_PER_TILE)])


def _sc_degree(rowd):
    mesh = plsc.VectorSubcoreMesh(core_axis_name="c", subcore_axis_name="s",
                                  num_cores=NC, num_subcores=NS)
    f = pl.kernel(
        _deg_body,
        out_type=jax.ShapeDtypeStruct((NC, N_PAD), jnp.float32),
        mesh=mesh,
        scratch_types=[
            pltpu.VMEM_SHARED((N_PAD,), jnp.float32),
            pltpu.VMEM((CPWD, CHD), jnp.int32),
            pltpu.VMEM((CHD,), jnp.float32),
            pltpu.VMEM((ROWS_PER_TILE,), jnp.float32),
            pltpu.SemaphoreType.DMA,
        ],
        compiler_params=pltpu.CompilerParams(needs_layout_passes=False),
    )
    return f(rowd)


def _edge_body(rowg_hbm, col_hbm, ea_hbm, y_hbm,
               agg_hbm,
               sh_agg, gb0, gb1, gb2, gb3,
               ridx4, cidx4, ea4,
               semi, semg, semsc):
    c = lax.axis_index("c")
    s = lax.axis_index("s")
    wid = c * NS + s
    gbufs = (gb0, gb1, gb2, gb3)
    cbase = wid * CPW

    def _zero_gb0(i, _):
        for j in range(D // L):
            gb0[i, pl.ds(j * L, L)] = _zero_vec16()
        return 0
    lax.fori_loop(0, CH, _zero_gb0, 0)

    base_n = s * ROWS_PER_TILE
    for t in range(ROWS_PER_TILE // CH):
        pltpu.sync_copy(gb0, sh_agg.at[pl.ds(base_n + t * CH, CH)])
    plsc.subcore_barrier()

    def _start_idx(slot, k):
        pltpu.async_copy(rowg_hbm.at[cbase + k], ridx4.at[slot], semi.at[slot])
        pltpu.async_copy(col_hbm.at[cbase + k], cidx4.at[slot], semi.at[slot])
        pltpu.async_copy(ea_hbm.at[cbase + k], ea4.at[slot], semi.at[slot])

    def _wait_idx(slot, k):
        pltpu.make_async_copy(rowg_hbm.at[cbase + k], ridx4.at[slot],
                              semi.at[slot]).wait()
        pltpu.make_async_copy(col_hbm.at[cbase + k], cidx4.at[slot],
                              semi.at[slot]).wait()
        pltpu.make_async_copy(ea_hbm.at[cbase + k], ea4.at[slot],
                              semi.at[slot]).wait()

    def _start_gather(slot, k):
        pltpu.async_copy(y_hbm.at[ridx4.at[slot]], gbufs[slot], semg.at[slot])

    def _wait_gather(slot):
        pltpu.make_async_copy(y_hbm.at[ridx4.at[slot]], gbufs[slot],
                              semg.at[slot]).wait()

    def _start_scatter(slot):
        pltpu.async_copy(gbufs[slot], sh_agg.at[cidx4.at[slot]],
                         semsc.at[slot], add=True)

    def _wait_scatter(slot):
        pltpu.make_async_copy(gbufs[slot], sh_agg.at[cidx4.at[slot]],
                              semsc.at[slot]).wait()

    _start_idx(0, 0)
    _start_idx(1, 1)
    _wait_idx(0, 0)
    _start_gather(0, 0)

    def _outer(o, _):
        for u in range(NBUF):
            i = o * NBUF + u
            nslot = (u + 1) % NBUF
            fslot = (u + 2) % NBUF

            @pl.when(i + 1 < CPW)
            def _():
                _wait_idx(nslot, i + 1)
                _start_gather(nslot, i + 1)

            _wait_gather(u)

            gb = gbufs[u]

            @plsc.parallel_loop(0, CH, 1, unroll=4)
            def _scale(e):
                sp = plsc.load_gather(
                    ea4, [jnp.full((L,), u, jnp.int32),
                          jnp.full((L,), e, jnp.int32)])
                for j in range(D // L):
                    sl = pl.ds(j * L, L)
                    gb[e, sl] = gb[e, sl] * sp

            _start_scatter(u)

            @pl.when(i >= 2)
            def _():
                _wait_scatter(fslot)

            @pl.when(i + 2 < CPW)
            def _():
                _start_idx(fslot, i + 2)
        return 0
    lax.fori_loop(0, CPW // NBUF, _outer, 0)

    _wait_scatter((CPW - 2) % NBUF)
    _wait_scatter((CPW - 1) % NBUF)
    plsc.subcore_barrier()

    pltpu.sync_copy(sh_agg.at[pl.ds(base_n, ROWS_PER_TILE)],
                    agg_hbm.at[c, pl.ds(base_n, ROWS_PER_TILE)])


def _sc_edges(rowg, col, ea, y):
    mesh = plsc.VectorSubcoreMesh(core_axis_name="c", subcore_axis_name="s",
                                  num_cores=NC, num_subcores=NS)
    f = pl.kernel(
        _edge_body,
        out_type=jax.ShapeDtypeStruct((NC, N_PAD, D), jnp.float32),
        mesh=mesh,
        scratch_types=[
            pltpu.VMEM_SHARED((N_PAD, D), jnp.float32),
            pltpu.VMEM((CH, D), jnp.float32),
            pltpu.VMEM((CH, D), jnp.float32),
            pltpu.VMEM((CH, D), jnp.float32),
            pltpu.VMEM((CH, D), jnp.float32),
            pltpu.VMEM((NBUF, CH), jnp.int32),
            pltpu.VMEM((NBUF, CH), jnp.int32),
            pltpu.VMEM((NBUF, CH), jnp.float32),
            pltpu.SemaphoreType.DMA((NBUF,)),
            pltpu.SemaphoreType.DMA((NBUF,)),
            pltpu.SemaphoreType.DMA((NBUF,)),
        ],
        compiler_params=pltpu.CompilerParams(needs_layout_passes=False),
    )
    return f(rowg, col, ea, y)


_BM = 1000


def _matmul_body(x_ref, w_ref, deg_ref, yp_ref, dis_ref):
    d = lax.rsqrt(deg_ref[0] + deg_ref[1] + 1.0)
    yp_ref[...] = jnp.dot(x_ref[...], w_ref[...],
                          preferred_element_type=jnp.float32) * d
    dis_ref[...] = d


def _tc_matmul(x, W1, deg2):
    return pl.pallas_call(
        _matmul_body,
        grid=(N_NODES // _BM,),
        in_specs=[pl.BlockSpec((_BM, D), lambda i: (i, 0)),
                  pl.BlockSpec((D, D), lambda i: (0, 0)),
                  pl.BlockSpec((NC, _BM, 1), lambda i: (0, i, 0))],
        out_specs=(pl.BlockSpec((_BM, D), lambda i: (i, 0)),
                   pl.BlockSpec((_BM, 1), lambda i: (i, 0))),
        out_shape=(jax.ShapeDtypeStruct((N_NODES, D), jnp.float32),
                   jax.ShapeDtypeStruct((N_NODES, 1), jnp.float32)),
    )(x, W1, deg2)


def _epilogue_body(a0_ref, a1_ref, yp_ref, d_ref, b1_ref, wl_ref, bl_ref, o_ref):
    a = (a0_ref[...] + a1_ref[...] + yp_ref[...]) * d_ref[...] + b1_ref[...]
    h = jnp.tanh(a)
    o_ref[...] = jax.nn.sigmoid(
        jnp.dot(h, wl_ref[...], preferred_element_type=jnp.float32)
        + bl_ref[0, 0])


def _tc_epilogue(agg, yp, dis, b1, Wl, bl):
    a0 = agg[0]
    a1 = agg[1]
    b1r = b1[None, :]
    blr = bl[None, :]
    return pl.pallas_call(
        _epilogue_body,
        grid=(N_NODES // _BM,),
        in_specs=[pl.BlockSpec((_BM, D), lambda i: (i, 0)),
                  pl.BlockSpec((_BM, D), lambda i: (i, 0)),
                  pl.BlockSpec((_BM, D), lambda i: (i, 0)),
                  pl.BlockSpec((_BM, 1), lambda i: (i, 0)),
                  pl.BlockSpec((1, D), lambda i: (0, 0)),
                  pl.BlockSpec((D, 1), lambda i: (0, 0)),
                  pl.BlockSpec((1, 1), lambda i: (0, 0))],
        out_specs=pl.BlockSpec((_BM, 1), lambda i: (i, 0)),
        out_shape=jax.ShapeDtypeStruct((N_NODES, 1), jnp.float32),
    )(a0, a1, yp, dis, b1r, Wl, blr)


def kernel(x, edge_index, edge_attr, W1, b1, Wl, bl):
    x = x.astype(jnp.float32)
    ei = edge_index.astype(jnp.int32)
    row = ei[0]
    col = ei[1]
    ea = edge_attr.astype(jnp.float32)

    npad_d = NCHD * CHD - E
    arp_d = jnp.arange(npad_d, dtype=jnp.int32)
    rowd = jnp.concatenate(
        [row, N_NODES + (arp_d % (N_PAD - N_NODES))]).reshape(NCHD, CHD)

    npad = NCH * CH - E
    arp = jnp.arange(npad, dtype=jnp.int32)
    pad_hi = N_NODES + (arp % (N_PAD - N_NODES))
    rowg = jnp.concatenate([row, arp % N_NODES]).reshape(NCH, CH)
    colp = jnp.concatenate([col, pad_hi]).reshape(NCH, CH)
    eap = jnp.concatenate([ea, jnp.zeros((npad,), jnp.float32)]).reshape(NCH, CH)

    deg2 = _sc_degree(rowd)
    yp, dis = _tc_matmul(x, W1.astype(jnp.float32),
                         deg2[:, :N_NODES, None])
    agg = _sc_edges(rowg, colp, eap, yp)
    out = _tc_epilogue(agg, yp, dis, b1.astype(jnp.float32),
                       Wl.astype(jnp.float32), bl.astype(jnp.float32))
    return out
